# baseline (speedup 1.0000x reference)
import jax
import jax.numpy as jnp
from jax import lax
from jax.experimental import pallas as pl
from jax.experimental.pallas import tpu as pltpu
from jax.experimental.pallas import tpu_sc as plsc

L = 2048
H = 16
T = 2 * L - 1
W = 3968
NG = 16
GM = 8


def _body(c3_hbm, out_hbm, sem):
    nc = 2
    c = lax.axis_index("c")
    s = lax.axis_index("s")
    wid = s * nc + c
    h = wid // nc
    half = wid % nc
    bi0 = half * (L // 2 // 8)
    xb = (1 - half) * 1024

    def fire(g):
        for m in range(GM):
            bi = bi0 + g + NG * m
            pltpu.async_copy(
                c3_hbm.at[
                    h,
                    pl.ds(pl.multiple_of(8 * g, 8), 8),
                    pl.ds(pl.multiple_of(xb + 896 - 128 * m, 128), L),
                ],
                out_hbm.at[0, h, pl.ds(pl.multiple_of(8 * bi, 8), 8), :],
                sem,
            )

    def drain():
        for _m in range(GM):
            pltpu.make_async_copy(
                c3_hbm.at[0, pl.ds(0, 8), pl.ds(0, L)],
                out_hbm.at[0, 0, pl.ds(0, 8), :],
                sem,
            ).wait()

    fire(0)

    def step(g, _):
        fire(g + 1)
        drain()
        return _

    lax.fori_loop(0, NG - 1, step, None)
    drain()


@jax.jit
def _run(rel_bias):
    rcp_t = rel_bias[::-1].T
    c3 = jnp.stack(
        [rcp_t[:, 127 - q : 127 - q + W] for q in range(128)], axis=1
    )
    k = pl.kernel(
        _body,
        mesh=plsc.VectorSubcoreMesh(core_axis_name="c", subcore_axis_name="s"),
        out_type=jax.ShapeDtypeStruct((1, H, L, L), jnp.float32),
        scratch_types=[
            pltpu.SemaphoreType.DMA,
        ],
    )
    return k(c3)


def kernel(rel_bias):
    return _run(rel_bias)


# speedup vs baseline: 22.3534x; 22.3534x over previous
import jax
import jax.numpy as jnp
from jax import lax
from jax.experimental import pallas as pl
from jax.experimental.pallas import tpu as pltpu
from jax.experimental.pallas import tpu_sc as plsc

L = 2048
H = 16
T = 2 * L - 1
W = 3968
PW = 2944
NG = 16
GM = 8


def _body(c3_hbm, out_hbm, p0_v, p1_v, sem_i0, sem_i1, sem_o):
    nc = 2
    c = lax.axis_index("c")
    s = lax.axis_index("s")
    wid = s * nc + c
    h = wid // nc
    half = wid % nc
    bi0 = half * (L // 2 // 8)
    xb = (1 - half) * 1024

    planes = (p0_v, p1_v)
    sems_i = (sem_i0, sem_i1)

    def load_plane(g, p):
        g = lax.rem(g, NG)
        pltpu.async_copy(
            c3_hbm.at[
                h,
                pl.ds(pl.multiple_of(8 * g, 8), 8),
                pl.ds(pl.multiple_of(xb, 128), PW),
            ],
            planes[p],
            sems_i[p],
        )

    def fire_group(g, p):
        for m in range(GM):
            bi = bi0 + g + NG * m
            pltpu.async_copy(
                planes[p].at[:, pl.ds(896 - 128 * m, L)],
                out_hbm.at[0, h, pl.ds(pl.multiple_of(8 * bi, 8), 8), :],
                sem_o,
            )

    def drain_group(p):
        for _m in range(GM):
            pltpu.make_async_copy(
                planes[p].at[:, pl.ds(0, L)],
                out_hbm.at[0, 0, pl.ds(0, 8), :],
                sem_o,
            ).wait()

    load_plane(0, 0)
    load_plane(1, 1)

    def step(gg, _):
        for p in range(2):
            g = 2 * gg + p
            pltpu.make_async_copy(c3_hbm.at[0, pl.ds(0, 8), pl.ds(0, PW)],
                                  planes[p], sems_i[p]).wait()
            fire_group(g, p)
            drain_group(p)
            load_plane(g + 2, p)
        return _

    lax.fori_loop(0, NG // 2, step, None)
    for p in range(2):
        pltpu.make_async_copy(c3_hbm.at[0, pl.ds(0, 8), pl.ds(0, PW)],
                              planes[p], sems_i[p]).wait()


@jax.jit
def _run(rel_bias):
    rcp_t = rel_bias[::-1].T
    c3 = jnp.stack(
        [rcp_t[:, 127 - q : 127 - q + W] for q in range(128)], axis=1
    )
    k = pl.kernel(
        _body,
        mesh=plsc.VectorSubcoreMesh(core_axis_name="c", subcore_axis_name="s"),
        out_type=jax.ShapeDtypeStruct((1, H, L, L), jnp.float32),
        scratch_types=[
            pltpu.VMEM((8, PW), jnp.float32),
            pltpu.VMEM((8, PW), jnp.float32),
            pltpu.SemaphoreType.DMA,
            pltpu.SemaphoreType.DMA,
            pltpu.SemaphoreType.DMA,
        ],
    )
    return k(c3)


def kernel(rel_bias):
    return _run(rel_bias)


# speedup vs baseline: 49.5647x; 2.2173x over previous
import jax
import jax.numpy as jnp
from jax import lax
from jax.experimental import pallas as pl
from jax.experimental.pallas import tpu as pltpu
from jax.experimental.pallas import tpu_sc as plsc

L = 2048
H = 16
T = 2 * L - 1
W = 3968
PW = 2944
NG = 16
GM = 8


def _body(c3_hbm, out_hbm, p0_v, p1_v, sem_i0, sem_i1, sem_o):
    nc = 2
    c = lax.axis_index("c")
    s = lax.axis_index("s")
    wid = s * nc + c
    h = wid // nc
    half = wid % nc
    bi0 = half * (L // 2 // 8)
    xb = (1 - half) * 1024

    planes = (p0_v, p1_v)
    sems_i = (sem_i0, sem_i1)

    def load_plane(g, p):
        g = lax.rem(g, NG)
        pltpu.async_copy(
            c3_hbm.at[
                h,
                pl.ds(pl.multiple_of(8 * g, 8), 8),
                pl.ds(pl.multiple_of(xb, 128), PW),
            ],
            planes[p],
            sems_i[p],
        )

    def fire_group(g, p):
        for m in range(GM):
            bi = bi0 + g + NG * m
            pltpu.async_copy(
                planes[p].at[:, pl.ds(896 - 128 * m, L)],
                out_hbm.at[0, h, pl.ds(pl.multiple_of(8 * bi, 8), 8), :],
                sem_o,
            )

    def drain_group(p):
        for _m in range(GM):
            pltpu.make_async_copy(
                planes[p].at[:, pl.ds(0, L)],
                out_hbm.at[0, 0, pl.ds(0, 8), :],
                sem_o,
            ).wait()

    load_plane(0, 0)
    load_plane(1, 1)

    def step(gg, _):
        for p in range(2):
            g = 2 * gg + p
            pltpu.make_async_copy(c3_hbm.at[0, pl.ds(0, 8), pl.ds(0, PW)],
                                  planes[p], sems_i[p]).wait()
            fire_group(g, p)
            drain_group(p)
            load_plane(g + 2, p)
        return _

    lax.fori_loop(0, NG // 2, step, None)
    for p in range(2):
        pltpu.make_async_copy(c3_hbm.at[0, pl.ds(0, 8), pl.ds(0, PW)],
                              planes[p], sems_i[p]).wait()


def _tc_build(a_ref, c3_ref):
    g = pl.program_id(0)
    rolled = pltpu.roll(a_ref[...], 8 * g - 120, axis=1)
    c3_ref[...] = rolled[:, :W].reshape(H, 8, W)


@jax.jit
def _run(rel_bias):
    rcp_t = jnp.pad(rel_bias[::-1].T, ((0, 0), (0, 9)))
    slab = jnp.stack(
        [rcp_t[:, 7 - r : 7 - r + 4096] for r in range(8)], axis=1
    ).reshape(128, 4096)
    c3 = pl.pallas_call(
        _tc_build,
        grid=(NG,),
        in_specs=[pl.BlockSpec((128, 4096), lambda g: (0, 0))],
        out_specs=pl.BlockSpec((H, 8, W), lambda g: (0, g, 0)),
        out_shape=jax.ShapeDtypeStruct((H, 128, W), jnp.float32),
    )(slab)
    k = pl.kernel(
        _body,
        mesh=plsc.VectorSubcoreMesh(core_axis_name="c", subcore_axis_name="s"),
        out_type=jax.ShapeDtypeStruct((1, H, L, L), jnp.float32),
        scratch_types=[
            pltpu.VMEM((8, PW), jnp.float32),
            pltpu.VMEM((8, PW), jnp.float32),
            pltpu.SemaphoreType.DMA,
            pltpu.SemaphoreType.DMA,
            pltpu.SemaphoreType.DMA,
        ],
    )
    return k(c3)


def kernel(rel_bias):
    return _run(rel_bias)
